# Initial kernel scaffold; baseline (speedup 1.0000x reference)
import jax
import jax.numpy as jnp
from jax.experimental import pallas as pl


def kernel(x, edge_index, W1a, b1a, W1b, b1b, bn1_g, bn1_b, W2a, b2a, W2b, b2b, bn2_g, bn2_b, g, A_k, D, Kindices, de, M, I):
    raise NotImplementedError("write your pallas kernel here")


# speedup vs baseline: 6.3572x; 6.3572x over previous
import functools

import jax
import jax.numpy as jnp
from jax import lax
from jax.experimental import pallas as pl
from jax.experimental.pallas import tpu as pltpu
from jax.experimental.pallas import tpu_sc as plsc

N = 10000
E = 320000
HID = 128
NCLS = 64
BN_EPS = 1e-5

NC = 2
NS = 16
NW = NC * NS
EPW = E // NW
CHUNK = 80
NCHUNK = EPW // CHUNK
RPT = 624
TAIL0 = NS * RPT
TAIL = N - TAIL0

SELU_ALPHA = 1.6732632423543772
SELU_SCALE = 1.0507009873554805


def _selu(z):
    return SELU_SCALE * jnp.where(z > 0, z, SELU_ALPHA * (jnp.exp(z) - 1.0))


def _make_agg(D):
    mesh = plsc.VectorSubcoreMesh(core_axis_name="c", subcore_axis_name="s")

    @functools.partial(
        pl.kernel,
        out_type=jax.ShapeDtypeStruct((NC, N, D), jnp.float32),
        mesh=mesh,
        scratch_types=[
            pltpu.VMEM((NCHUNK, CHUNK), jnp.int32),
            pltpu.VMEM((NCHUNK, CHUNK), jnp.int32),
            pltpu.VMEM((CHUNK, D), jnp.float32),
            pltpu.VMEM_SHARED((N, D), jnp.float32),
            pltpu.SemaphoreType.DMA,
        ],
    )
    def agg(h_hbm, src_hbm, dst_hbm, zeros_hbm, out_hbm,
            src_v, dst_v, rows_v, acc, sem):
        c = lax.axis_index("c")
        s = lax.axis_index("s")
        wid = c * NS + s
        row0 = s * RPT
        pltpu.sync_copy(zeros_hbm.at[pl.ds(row0, RPT)],
                        acc.at[pl.ds(row0, RPT)])

        @pl.when(s == NS - 1)
        def _():
            pltpu.sync_copy(zeros_hbm.at[pl.ds(TAIL0, TAIL)],
                            acc.at[pl.ds(TAIL0, TAIL)])
        pltpu.sync_copy(src_hbm.at[wid], src_v)
        pltpu.sync_copy(dst_hbm.at[wid], dst_v)
        plsc.subcore_barrier()

        def body(j, _):
            pltpu.async_copy(h_hbm.at[src_v.at[j]], rows_v, sem).wait()
            pltpu.sync_copy(rows_v, acc.at[dst_v.at[j]], add=True)
            return _

        lax.fori_loop(0, NCHUNK, body, 0)
        plsc.subcore_barrier()
        pltpu.sync_copy(acc.at[pl.ds(row0, RPT)],
                        out_hbm.at[c].at[pl.ds(row0, RPT)])

        @pl.when(s == NS - 1)
        def _():
            pltpu.sync_copy(acc.at[pl.ds(TAIL0, TAIL)],
                            out_hbm.at[c].at[pl.ds(TAIL0, TAIL)])

    return agg


def _dense1(x, p0, p1, W1a, b1a, W1b, b1b, g1, be1):
    def body(x_ref, p0_ref, p1_ref, wa, ba, wb, bb, gg, bb2, out_ref):
        z = x_ref[...] + p0_ref[...] + p1_ref[...]
        z = jnp.dot(z, wa[...], preferred_element_type=jnp.float32) + ba[...]
        z = jnp.maximum(z, 0.0)
        z = jnp.dot(z, wb[...], preferred_element_type=jnp.float32) + bb[...]
        h = _selu(z)
        mean = jnp.mean(h, axis=0, keepdims=True)
        var = jnp.mean((h - mean) ** 2, axis=0, keepdims=True)
        out_ref[...] = gg[...] * (h - mean) * lax.rsqrt(var + BN_EPS) + bb2[...]

    return pl.pallas_call(
        body,
        out_shape=jax.ShapeDtypeStruct((N, HID), jnp.float32),
    )(x, p0, p1, W1a, b1a, W1b, b1b, g1, be1)


def _dense2(h, p0, p1, W2a, b2a, W2b, b2b, g2, be2):
    def body(h_ref, p0_ref, p1_ref, wa, ba, wb, bb, gg, bb2, out_ref):
        z = h_ref[...] + p0_ref[...] + p1_ref[...]
        z = jnp.dot(z, wa[...], preferred_element_type=jnp.float32) + ba[...]
        z = jnp.maximum(z, 0.0)
        z = jnp.dot(z, wb[...], preferred_element_type=jnp.float32) + bb[...]
        h2 = _selu(z)
        mean = jnp.mean(h2, axis=0, keepdims=True)
        var = jnp.mean((h2 - mean) ** 2, axis=0, keepdims=True)
        h2 = gg[...] * (h2 - mean) * lax.rsqrt(var + BN_EPS) + bb2[...]
        m = jnp.max(h2, axis=1, keepdims=True)
        e = jnp.exp(h2 - m)
        out_ref[...] = e / jnp.sum(e, axis=1, keepdims=True)

    return pl.pallas_call(
        body,
        out_shape=jax.ShapeDtypeStruct((N, NCLS), jnp.float32),
    )(h, p0, p1, W2a, b2a, W2b, b2b, g2, be2)


def kernel(x, edge_index, W1a, b1a, W1b, b1b, bn1_g, bn1_b,
           W2a, b2a, W2b, b2b, bn2_g, bn2_b,
           g, A_k, D, Kindices, de, M, I):
    src = edge_index[0].astype(jnp.int32).reshape(NW, NCHUNK, CHUNK)
    dst = edge_index[1].astype(jnp.int32).reshape(NW, NCHUNK, CHUNK)
    zeros = jnp.zeros((N, HID), dtype=jnp.float32)

    agg = _make_agg(HID)

    b1a_ = b1a.reshape(1, HID)
    b1b_ = b1b.reshape(1, HID)
    g1_ = bn1_g.reshape(1, HID)
    be1_ = bn1_b.reshape(1, HID)
    b2a_ = b2a.reshape(1, HID)
    b2b_ = b2b.reshape(1, NCLS)
    g2_ = bn2_g.reshape(1, NCLS)
    be2_ = bn2_b.reshape(1, NCLS)

    p = agg(x, src, dst, zeros)
    h = _dense1(x, p[0], p[1], W1a, b1a_, W1b, b1b_, g1_, be1_)
    p2 = agg(h, src, dst, zeros)
    out = _dense2(h, p2[0], p2[1], W2a, b2a_, W2b, b2b_, g2_, be2_)
    return out


# speedup vs baseline: 10.7194x; 1.6862x over previous
import functools

import jax
import jax.numpy as jnp
from jax import lax
from jax.experimental import pallas as pl
from jax.experimental.pallas import tpu as pltpu
from jax.experimental.pallas import tpu_sc as plsc

N = 10000
E = 320000
HID = 128
NCLS = 64
BN_EPS = 1e-5

NC = 2
NS = 16
NW = NC * NS
EPW = E // NW
CHUNK = 50
NCHUNK = EPW // CHUNK
NBUF = 5
IDN = 2 * NBUF
RPT = 624
TAIL0 = NS * RPT
TAIL = N - TAIL0

SELU_ALPHA = 1.6732632423543772
SELU_SCALE = 1.0507009873554805


def _selu(z):
    return SELU_SCALE * jnp.where(z > 0, z, SELU_ALPHA * (jnp.exp(z) - 1.0))


def _make_agg(D):
    mesh = plsc.VectorSubcoreMesh(core_axis_name="c", subcore_axis_name="s")

    @functools.partial(
        pl.kernel,
        out_type=jax.ShapeDtypeStruct((NC, N, D), jnp.float32),
        mesh=mesh,
        scratch_types=[
            pltpu.VMEM((IDN, 2, CHUNK), jnp.int32),
            pltpu.VMEM((NBUF, CHUNK, D), jnp.float32),
            pltpu.VMEM_SHARED((N, D), jnp.float32),
        ] + [pltpu.SemaphoreType.DMA] * (NBUF + IDN),
    )
    def agg(h_hbm, ids_hbm, zeros_hbm, out_hbm,
            ids_v, rows_v, acc, *sems):
        gsem = sems[:NBUF]
        isem = sems[NBUF:]
        c = lax.axis_index("c")
        s = lax.axis_index("s")
        wid = c * NS + s
        row0 = s * RPT

        def id_fetch(j, slot):
            pltpu.async_copy(ids_hbm.at[wid].at[j], ids_v.at[slot], isem[slot])

        def id_wait(j, slot):
            pltpu.make_async_copy(ids_hbm.at[wid].at[j], ids_v.at[slot],
                                  isem[slot]).wait()

        def gather_start(j, slot, idslot):
            pltpu.async_copy(h_hbm.at[ids_v.at[idslot, 0]], rows_v.at[slot],
                             gsem[slot])

        def gather_wait(j, slot, idslot):
            pltpu.make_async_copy(h_hbm.at[ids_v.at[idslot, 0]],
                                  rows_v.at[slot], gsem[slot]).wait()

        pltpu.sync_copy(zeros_hbm.at[pl.ds(row0, RPT)],
                        acc.at[pl.ds(row0, RPT)])

        @pl.when(s == NS - 1)
        def _():
            pltpu.sync_copy(zeros_hbm.at[pl.ds(TAIL0, TAIL)],
                            acc.at[pl.ds(TAIL0, TAIL)])

        for bb in range(IDN):
            id_fetch(bb, bb)
        plsc.subcore_barrier()
        for b in range(NBUF):
            id_wait(b, b)
            gather_start(b, b, b)

        def body(jo, carry):
            for bb in range(IDN):
                j = jo * IDN + bb
                b = bb % NBUF
                gather_wait(j, b, bb)
                pltpu.sync_copy(rows_v.at[b], acc.at[ids_v.at[bb, 1]], add=True)
                @pl.when(j + IDN < NCHUNK)
                def _idrefill():
                    id_fetch(j + IDN, bb)
                @pl.when(j + NBUF < NCHUNK)
                def _refill():
                    nb = (bb + NBUF) % IDN
                    id_wait(j + NBUF, nb)
                    gather_start(j + NBUF, b, nb)
            return carry

        lax.fori_loop(0, NCHUNK // IDN, body, 0)
        plsc.subcore_barrier()
        pltpu.sync_copy(acc.at[pl.ds(row0, RPT)],
                        out_hbm.at[c].at[pl.ds(row0, RPT)])

        @pl.when(s == NS - 1)
        def _():
            pltpu.sync_copy(acc.at[pl.ds(TAIL0, TAIL)],
                            out_hbm.at[c].at[pl.ds(TAIL0, TAIL)])

    return agg


def _dense1(x, p0, p1, W1a, b1a, W1b, b1b, g1, be1):
    def body(x_ref, p0_ref, p1_ref, wa, ba, wb, bb, gg, bb2, out_ref):
        z = x_ref[...] + p0_ref[...] + p1_ref[...]
        z = jnp.dot(z, wa[...], preferred_element_type=jnp.float32) + ba[...]
        z = jnp.maximum(z, 0.0)
        z = jnp.dot(z, wb[...], preferred_element_type=jnp.float32) + bb[...]
        h = _selu(z)
        mean = jnp.mean(h, axis=0, keepdims=True)
        var = jnp.mean((h - mean) ** 2, axis=0, keepdims=True)
        out_ref[...] = gg[...] * (h - mean) * lax.rsqrt(var + BN_EPS) + bb2[...]

    return pl.pallas_call(
        body,
        out_shape=jax.ShapeDtypeStruct((N, HID), jnp.float32),
    )(x, p0, p1, W1a, b1a, W1b, b1b, g1, be1)


def _dense2(h, p0, p1, W2a, b2a, W2b, b2b, g2, be2):
    def body(h_ref, p0_ref, p1_ref, wa, ba, wb, bb, gg, bb2, out_ref):
        z = h_ref[...] + p0_ref[...] + p1_ref[...]
        z = jnp.dot(z, wa[...], preferred_element_type=jnp.float32) + ba[...]
        z = jnp.maximum(z, 0.0)
        z = jnp.dot(z, wb[...], preferred_element_type=jnp.float32) + bb[...]
        h2 = _selu(z)
        mean = jnp.mean(h2, axis=0, keepdims=True)
        var = jnp.mean((h2 - mean) ** 2, axis=0, keepdims=True)
        h2 = gg[...] * (h2 - mean) * lax.rsqrt(var + BN_EPS) + bb2[...]
        m = jnp.max(h2, axis=1, keepdims=True)
        e = jnp.exp(h2 - m)
        out_ref[...] = e / jnp.sum(e, axis=1, keepdims=True)

    return pl.pallas_call(
        body,
        out_shape=jax.ShapeDtypeStruct((N, NCLS), jnp.float32),
    )(h, p0, p1, W2a, b2a, W2b, b2b, g2, be2)


def kernel(x, edge_index, W1a, b1a, W1b, b1b, bn1_g, bn1_b,
           W2a, b2a, W2b, b2b, bn2_g, bn2_b,
           g, A_k, D, Kindices, de, M, I):
    ei = edge_index.astype(jnp.int32).reshape(2, NW, NCHUNK, CHUNK)
    ids = jnp.stack([ei[0], ei[1]], axis=2)
    zeros = jnp.zeros((N, HID), dtype=jnp.float32)

    agg = _make_agg(HID)

    b1a_ = b1a.reshape(1, HID)
    b1b_ = b1b.reshape(1, HID)
    g1_ = bn1_g.reshape(1, HID)
    be1_ = bn1_b.reshape(1, HID)
    b2a_ = b2a.reshape(1, HID)
    b2b_ = b2b.reshape(1, NCLS)
    g2_ = bn2_g.reshape(1, NCLS)
    be2_ = bn2_b.reshape(1, NCLS)

    p = agg(x, ids, zeros)
    h = _dense1(x, p[0], p[1], W1a, b1a_, W1b, b1b_, g1_, be1_)
    p2 = agg(h, ids, zeros)
    out = _dense2(h, p2[0], p2[1], W2a, b2a_, W2b, b2b_, g2_, be2_)
    return out


# speedup vs baseline: 12.2131x; 1.1393x over previous
import functools

import jax
import jax.numpy as jnp
from jax import lax
from jax.experimental import pallas as pl
from jax.experimental.pallas import tpu as pltpu
from jax.experimental.pallas import tpu_sc as plsc

N = 10000
E = 320000
HID = 128
NCLS = 64
BN_EPS = 1e-5

NC = 2
NS = 16
NW = NC * NS
EPW = E // NW
CHUNK = 50
NCHUNK = EPW // CHUNK
RN = 5
NBUF = 4
IDN = 10
RPT = 624
TAIL0 = NS * RPT
TAIL = N - TAIL0

SELU_ALPHA = 1.6732632423543772
SELU_SCALE = 1.0507009873554805


def _selu(z):
    return SELU_SCALE * jnp.where(z > 0, z, SELU_ALPHA * (jnp.exp(z) - 1.0))


def _make_agg(D):
    mesh = plsc.VectorSubcoreMesh(core_axis_name="c", subcore_axis_name="s")

    @functools.partial(
        pl.kernel,
        out_type=jax.ShapeDtypeStruct((NC, N, D), jnp.float32),
        mesh=mesh,
        scratch_types=[
            pltpu.VMEM((IDN, 2, 1, CHUNK), jnp.int32),
            pltpu.VMEM((RN, CHUNK, D), jnp.float32),
            pltpu.VMEM_SHARED((N, D), jnp.float32),
        ] + [pltpu.SemaphoreType.DMA] * (2 * RN + IDN),
    )
    def agg(h_hbm, ids_hbm, zeros_hbm, out_hbm,
            ids_v, rows_v, acc, *sems):
        gsem = sems[:RN]
        ssem = sems[RN:2 * RN]
        isem = sems[2 * RN:]
        c = lax.axis_index("c")
        s = lax.axis_index("s")
        wid = c * NS + s
        row0 = s * RPT

        def id_fetch(j, slot):
            pltpu.async_copy(ids_hbm.at[0, wid, j], ids_v.at[slot, 0], isem[slot])
            pltpu.async_copy(ids_hbm.at[1, wid, j], ids_v.at[slot, 1], isem[slot])

        def id_wait(j, slot):
            pltpu.make_async_copy(ids_hbm.at[0, wid, j], ids_v.at[slot, 0],
                                  isem[slot]).wait()
            pltpu.make_async_copy(ids_hbm.at[1, wid, j], ids_v.at[slot, 1],
                                  isem[slot]).wait()

        def gather_start(j, slot, idslot):
            pltpu.async_copy(h_hbm.at[ids_v.at[idslot, 0, 0]], rows_v.at[slot],
                             gsem[slot])

        def gather_wait(j, slot, idslot):
            pltpu.make_async_copy(h_hbm.at[ids_v.at[idslot, 0, 0]],
                                  rows_v.at[slot], gsem[slot]).wait()

        def scatter_start(j, slot, idslot):
            pltpu.async_copy(rows_v.at[slot], acc.at[ids_v.at[idslot, 1, 0]],
                             ssem[slot], add=True)

        def scatter_wait(j, slot, idslot):
            pltpu.make_async_copy(rows_v.at[slot], acc.at[ids_v.at[idslot, 1, 0]],
                                  ssem[slot]).wait()

        pltpu.sync_copy(zeros_hbm.at[pl.ds(row0, RPT)],
                        acc.at[pl.ds(row0, RPT)])

        @pl.when(s == NS - 1)
        def _():
            pltpu.sync_copy(zeros_hbm.at[pl.ds(TAIL0, TAIL)],
                            acc.at[pl.ds(TAIL0, TAIL)])

        for bb in range(IDN):
            id_fetch(bb, bb)
        plsc.subcore_barrier()
        for b in range(NBUF):
            id_wait(b, b)
            gather_start(b, b, b)

        def body(jo, carry):
            for bb in range(IDN):
                j = jo * IDN + bb
                b = bb % RN
                gather_wait(j, b, bb)
                scatter_start(j, b, bb)

                @pl.when((j >= 1) & (j + NBUF < NCHUNK))
                def _drain_prev():
                    scatter_wait(j - 1, (b + RN - 1) % RN, (bb + IDN - 1) % IDN)

                @pl.when((j >= 1) & (j + IDN - 1 < NCHUNK))
                def _idrefill():
                    id_fetch(j + IDN - 1, (bb + IDN - 1) % IDN)

                @pl.when(j + NBUF < NCHUNK)
                def _refill():
                    nb = (bb + NBUF) % IDN
                    id_wait(j + NBUF, nb)
                    gather_start(j + NBUF, (b + NBUF) % RN, nb)
            return carry

        lax.fori_loop(0, NCHUNK // IDN, body, 0)
        for m in range(NCHUNK - RN, NCHUNK):
            scatter_wait(m, m % RN, m % IDN)
        plsc.subcore_barrier()
        pltpu.sync_copy(acc.at[pl.ds(row0, RPT)],
                        out_hbm.at[c].at[pl.ds(row0, RPT)])

        @pl.when(s == NS - 1)
        def _():
            pltpu.sync_copy(acc.at[pl.ds(TAIL0, TAIL)],
                            out_hbm.at[c].at[pl.ds(TAIL0, TAIL)])

    return agg


def _dense1(x, p, W1a, b1a, W1b, b1b, g1, be1):
    def body(x_ref, p_ref, wa, ba, wb, bb, gg, bb2, out_ref):
        z = x_ref[...] + p_ref[0] + p_ref[1]
        z = jnp.dot(z, wa[...], preferred_element_type=jnp.float32) + ba[...]
        z = jnp.maximum(z, 0.0)
        z = jnp.dot(z, wb[...], preferred_element_type=jnp.float32) + bb[...]
        h = _selu(z)
        mean = jnp.mean(h, axis=0, keepdims=True)
        var = jnp.mean((h - mean) ** 2, axis=0, keepdims=True)
        out_ref[...] = gg[...] * (h - mean) * lax.rsqrt(var + BN_EPS) + bb2[...]

    return pl.pallas_call(
        body,
        out_shape=jax.ShapeDtypeStruct((N, HID), jnp.float32),
    )(x, p, W1a, b1a, W1b, b1b, g1, be1)


def _dense2(h, p, W2a, b2a, W2b, b2b, g2, be2):
    def body(h_ref, p_ref, wa, ba, wb, bb, gg, bb2, out_ref):
        z = h_ref[...] + p_ref[0] + p_ref[1]
        z = jnp.dot(z, wa[...], preferred_element_type=jnp.float32) + ba[...]
        z = jnp.maximum(z, 0.0)
        z = jnp.dot(z, wb[...], preferred_element_type=jnp.float32) + bb[...]
        h2 = _selu(z)
        mean = jnp.mean(h2, axis=0, keepdims=True)
        var = jnp.mean((h2 - mean) ** 2, axis=0, keepdims=True)
        h2 = gg[...] * (h2 - mean) * lax.rsqrt(var + BN_EPS) + bb2[...]
        m = jnp.max(h2, axis=1, keepdims=True)
        e = jnp.exp(h2 - m)
        out_ref[...] = e / jnp.sum(e, axis=1, keepdims=True)

    return pl.pallas_call(
        body,
        out_shape=jax.ShapeDtypeStruct((N, NCLS), jnp.float32),
    )(h, p, W2a, b2a, W2b, b2b, g2, be2)


def kernel(x, edge_index, W1a, b1a, W1b, b1b, bn1_g, bn1_b,
           W2a, b2a, W2b, b2b, bn2_g, bn2_b,
           g, A_k, D, Kindices, de, M, I):
    ids = edge_index.astype(jnp.int32).reshape(2, NW, NCHUNK, 1, CHUNK)
    zeros = jnp.zeros((N, HID), dtype=jnp.float32)

    agg = _make_agg(HID)

    b1a_ = b1a.reshape(1, HID)
    b1b_ = b1b.reshape(1, HID)
    g1_ = bn1_g.reshape(1, HID)
    be1_ = bn1_b.reshape(1, HID)
    b2a_ = b2a.reshape(1, HID)
    b2b_ = b2b.reshape(1, NCLS)
    g2_ = bn2_g.reshape(1, NCLS)
    be2_ = bn2_b.reshape(1, NCLS)

    p = agg(x, ids, zeros)
    h = _dense1(x, p, W1a, b1a_, W1b, b1b_, g1_, be1_)
    p2 = agg(h, ids, zeros)
    out = _dense2(h, p2, W2a, b2a_, W2b, b2b_, g2_, be2_)
    return out


# speedup vs baseline: 12.3379x; 1.0102x over previous
import functools

import jax
import jax.numpy as jnp
from jax import lax
from jax.experimental import pallas as pl
from jax.experimental.pallas import tpu as pltpu
from jax.experimental.pallas import tpu_sc as plsc

N = 10000
E = 320000
HID = 128
NCLS = 64
BN_EPS = 1e-5

NC = 2
NS = 16
NW = NC * NS
EPW = E // NW
CHUNK = 50
NCHUNK = EPW // CHUNK
RN = 5
NBUF = 4
IDN = 10
RPT = 624
TAIL0 = NS * RPT
TAIL = N - TAIL0

SELU_ALPHA = 1.6732632423543772
SELU_SCALE = 1.0507009873554805


def _selu(z):
    return SELU_SCALE * jnp.where(z > 0, z, SELU_ALPHA * (jnp.exp(z) - 1.0))


def _make_agg(D):
    mesh = plsc.VectorSubcoreMesh(core_axis_name="c", subcore_axis_name="s")

    @functools.partial(
        pl.kernel,
        out_type=jax.ShapeDtypeStruct((NC, N, D), jnp.float32),
        mesh=mesh,
        scratch_types=[
            pltpu.VMEM((IDN, 2, 1, CHUNK), jnp.int32),
            pltpu.VMEM((RN, CHUNK, D), jnp.float32),
            pltpu.VMEM_SHARED((N, D), jnp.float32),
        ] + [pltpu.SemaphoreType.DMA] * (2 * RN + IDN),
    )
    def agg(h_hbm, ids_hbm, zeros_hbm, out_hbm,
            ids_v, rows_v, acc, *sems):
        gsem = sems[:RN]
        ssem = sems[RN:2 * RN]
        isem = sems[2 * RN:]
        c = lax.axis_index("c")
        s = lax.axis_index("s")
        wid = c * NS + s
        row0 = s * RPT

        def id_fetch(j, slot):
            pltpu.async_copy(ids_hbm.at[0, wid, j], ids_v.at[slot, 0], isem[slot])
            pltpu.async_copy(ids_hbm.at[1, wid, j], ids_v.at[slot, 1], isem[slot])

        def id_wait(j, slot):
            pltpu.make_async_copy(ids_hbm.at[0, wid, j], ids_v.at[slot, 0],
                                  isem[slot]).wait()
            pltpu.make_async_copy(ids_hbm.at[1, wid, j], ids_v.at[slot, 1],
                                  isem[slot]).wait()

        def gather_start(j, slot, idslot):
            pltpu.async_copy(h_hbm.at[ids_v.at[idslot, 0, 0]], rows_v.at[slot],
                             gsem[slot])

        def gather_wait(j, slot, idslot):
            pltpu.make_async_copy(h_hbm.at[ids_v.at[idslot, 0, 0]],
                                  rows_v.at[slot], gsem[slot]).wait()

        def scatter_start(j, slot, idslot):
            pltpu.async_copy(rows_v.at[slot], acc.at[ids_v.at[idslot, 1, 0]],
                             ssem[slot], add=True)

        def scatter_wait(j, slot, idslot):
            pltpu.make_async_copy(rows_v.at[slot], acc.at[ids_v.at[idslot, 1, 0]],
                                  ssem[slot]).wait()

        def _init(src):
            pltpu.sync_copy(src.at[pl.ds(row0, RPT)], acc.at[pl.ds(row0, RPT)])

            @pl.when(s == NS - 1)
            def _tail():
                pltpu.sync_copy(src.at[pl.ds(TAIL0, TAIL)],
                                acc.at[pl.ds(TAIL0, TAIL)])

        @pl.when(c == 0)
        def _initx():
            _init(h_hbm)

        @pl.when(c == 1)
        def _initz():
            _init(zeros_hbm)

        for bb in range(IDN):
            id_fetch(bb, bb)
        plsc.subcore_barrier()
        for b in range(NBUF):
            id_wait(b, b)
            gather_start(b, b, b)

        def body(jo, carry):
            for bb in range(IDN):
                j = jo * IDN + bb
                b = bb % RN
                gather_wait(j, b, bb)
                scatter_start(j, b, bb)

                @pl.when((j >= 1) & (j + NBUF < NCHUNK))
                def _drain_prev():
                    scatter_wait(j - 1, (b + RN - 1) % RN, (bb + IDN - 1) % IDN)

                @pl.when((j >= 1) & (j + IDN - 1 < NCHUNK))
                def _idrefill():
                    id_fetch(j + IDN - 1, (bb + IDN - 1) % IDN)

                @pl.when(j + NBUF < NCHUNK)
                def _refill():
                    nb = (bb + NBUF) % IDN
                    id_wait(j + NBUF, nb)
                    gather_start(j + NBUF, (b + NBUF) % RN, nb)
            return carry

        lax.fori_loop(0, NCHUNK // IDN, body, 0)
        for m in range(NCHUNK - RN, NCHUNK):
            scatter_wait(m, m % RN, m % IDN)
        plsc.subcore_barrier()
        pltpu.sync_copy(acc.at[pl.ds(row0, RPT)],
                        out_hbm.at[c].at[pl.ds(row0, RPT)])

        @pl.when(s == NS - 1)
        def _():
            pltpu.sync_copy(acc.at[pl.ds(TAIL0, TAIL)],
                            out_hbm.at[c].at[pl.ds(TAIL0, TAIL)])

    return agg


def _dense1(p, W1a, b1a, W1b, b1b, g1, be1):
    def body(p_ref, wa, ba, wb, bb, gg, bb2, out_ref):
        z = p_ref[0] + p_ref[1]
        z = jnp.dot(z, wa[...], preferred_element_type=jnp.float32) + ba[...]
        z = jnp.maximum(z, 0.0)
        z = jnp.dot(z, wb[...], preferred_element_type=jnp.float32) + bb[...]
        h = _selu(z)
        mean = jnp.mean(h, axis=0, keepdims=True)
        var = jnp.mean((h - mean) ** 2, axis=0, keepdims=True)
        out_ref[...] = gg[...] * (h - mean) * lax.rsqrt(var + BN_EPS) + bb2[...]

    return pl.pallas_call(
        body,
        out_shape=jax.ShapeDtypeStruct((N, HID), jnp.float32),
    )(p, W1a, b1a, W1b, b1b, g1, be1)


def _dense2(p, W2a, b2a, W2b, b2b, g2, be2):
    def body(p_ref, wa, ba, wb, bb, gg, bb2, out_ref):
        z = p_ref[0] + p_ref[1]
        z = jnp.dot(z, wa[...], preferred_element_type=jnp.float32) + ba[...]
        z = jnp.maximum(z, 0.0)
        z = jnp.dot(z, wb[...], preferred_element_type=jnp.float32) + bb[...]
        h2 = _selu(z)
        mean = jnp.mean(h2, axis=0, keepdims=True)
        var = jnp.mean((h2 - mean) ** 2, axis=0, keepdims=True)
        h2 = gg[...] * (h2 - mean) * lax.rsqrt(var + BN_EPS) + bb2[...]
        m = jnp.max(h2, axis=1, keepdims=True)
        e = jnp.exp(h2 - m)
        out_ref[...] = e / jnp.sum(e, axis=1, keepdims=True)

    return pl.pallas_call(
        body,
        out_shape=jax.ShapeDtypeStruct((N, NCLS), jnp.float32),
    )(p, W2a, b2a, W2b, b2b, g2, be2)


def kernel(x, edge_index, W1a, b1a, W1b, b1b, bn1_g, bn1_b,
           W2a, b2a, W2b, b2b, bn2_g, bn2_b,
           g, A_k, D, Kindices, de, M, I):
    ids = edge_index.astype(jnp.int32).reshape(2, NW, NCHUNK, 1, CHUNK)
    zeros = jnp.zeros((N, HID), dtype=jnp.float32)

    agg = _make_agg(HID)

    b1a_ = b1a.reshape(1, HID)
    b1b_ = b1b.reshape(1, HID)
    g1_ = bn1_g.reshape(1, HID)
    be1_ = bn1_b.reshape(1, HID)
    b2a_ = b2a.reshape(1, HID)
    b2b_ = b2b.reshape(1, NCLS)
    g2_ = bn2_g.reshape(1, NCLS)
    be2_ = bn2_b.reshape(1, NCLS)

    p = agg(x, ids, zeros)
    h = _dense1(p, W1a, b1a_, W1b, b1b_, g1_, be1_)
    p2 = agg(h, ids, zeros)
    out = _dense2(p2, W2a, b2a_, W2b, b2b_, g2_, be2_)
    return out


# speedup vs baseline: 12.4324x; 1.0077x over previous
import functools

import jax
import jax.numpy as jnp
from jax import lax
from jax.experimental import pallas as pl
from jax.experimental.pallas import tpu as pltpu
from jax.experimental.pallas import tpu_sc as plsc

N = 10000
E = 320000
HID = 128
NCLS = 64
BN_EPS = 1e-5

NC = 2
NS = 16
NW = NC * NS
EPW = E // NW
CHUNK = 40
NCHUNK = EPW // CHUNK
RN = 5
NBUF = 4
IDN = 10
RPT = 624
TAIL0 = NS * RPT
TAIL = N - TAIL0

SELU_ALPHA = 1.6732632423543772
SELU_SCALE = 1.0507009873554805


def _selu(z):
    return SELU_SCALE * jnp.where(z > 0, z, SELU_ALPHA * (jnp.exp(z) - 1.0))


def _make_agg(D):
    mesh = plsc.VectorSubcoreMesh(core_axis_name="c", subcore_axis_name="s")

    @functools.partial(
        pl.kernel,
        out_type=jax.ShapeDtypeStruct((NC, N, D), jnp.float32),
        mesh=mesh,
        scratch_types=[
            pltpu.VMEM((IDN, 2, CHUNK), jnp.int32),
            pltpu.VMEM((RN, CHUNK, D), jnp.float32),
            pltpu.VMEM_SHARED((N, D), jnp.float32),
        ] + [pltpu.SemaphoreType.DMA] * (2 * RN + IDN),
    )
    def agg(h_hbm, srcf_hbm, dstf_hbm, zeros_hbm, out_hbm,
            ids_v, rows_v, acc, *sems):
        gsem = sems[:RN]
        ssem = sems[RN:2 * RN]
        isem = sems[2 * RN:]
        c = lax.axis_index("c")
        s = lax.axis_index("s")
        wid = c * NS + s
        row0 = s * RPT

        def _eoff(j):
            return pl.multiple_of(wid * EPW + j * CHUNK, 8)

        def id_fetch(j, slot):
            off = _eoff(j)
            pltpu.async_copy(srcf_hbm.at[pl.ds(off, CHUNK)], ids_v.at[slot, 0],
                             isem[slot])
            pltpu.async_copy(dstf_hbm.at[pl.ds(off, CHUNK)], ids_v.at[slot, 1],
                             isem[slot])

        def id_wait(j, slot):
            off = _eoff(j)
            pltpu.make_async_copy(srcf_hbm.at[pl.ds(off, CHUNK)],
                                  ids_v.at[slot, 0], isem[slot]).wait()
            pltpu.make_async_copy(dstf_hbm.at[pl.ds(off, CHUNK)],
                                  ids_v.at[slot, 1], isem[slot]).wait()

        def gather_start(j, slot, idslot):
            pltpu.async_copy(h_hbm.at[ids_v.at[idslot, 0]], rows_v.at[slot],
                             gsem[slot])

        def gather_wait(j, slot, idslot):
            pltpu.make_async_copy(h_hbm.at[ids_v.at[idslot, 0]],
                                  rows_v.at[slot], gsem[slot]).wait()

        def scatter_start(j, slot, idslot):
            pltpu.async_copy(rows_v.at[slot], acc.at[ids_v.at[idslot, 1]],
                             ssem[slot], add=True)

        def scatter_wait(j, slot, idslot):
            pltpu.make_async_copy(rows_v.at[slot], acc.at[ids_v.at[idslot, 1]],
                                  ssem[slot]).wait()

        def _init(src):
            pltpu.sync_copy(src.at[pl.ds(row0, RPT)], acc.at[pl.ds(row0, RPT)])

            @pl.when(s == NS - 1)
            def _tail():
                pltpu.sync_copy(src.at[pl.ds(TAIL0, TAIL)],
                                acc.at[pl.ds(TAIL0, TAIL)])

        for bb in range(IDN):
            id_fetch(bb, bb)

        @pl.when(c == 0)
        def _initx():
            _init(h_hbm)

        @pl.when(c == 1)
        def _initz():
            _init(zeros_hbm)

        for b in range(NBUF):
            id_wait(b, b)
            gather_start(b, b, b)
        plsc.subcore_barrier()

        def body(jo, carry):
            for bb in range(IDN):
                j = jo * IDN + bb
                b = bb % RN
                gather_wait(j, b, bb)
                scatter_start(j, b, bb)

                @pl.when((j >= 1) & (j + NBUF < NCHUNK))
                def _drain_prev():
                    scatter_wait(j - 1, (b + RN - 1) % RN, (bb + IDN - 1) % IDN)

                @pl.when((j >= 1) & (j + IDN - 1 < NCHUNK))
                def _idrefill():
                    id_fetch(j + IDN - 1, (bb + IDN - 1) % IDN)

                @pl.when(j + NBUF < NCHUNK)
                def _refill():
                    nb = (bb + NBUF) % IDN
                    id_wait(j + NBUF, nb)
                    gather_start(j + NBUF, (b + NBUF) % RN, nb)
            return carry

        lax.fori_loop(0, NCHUNK // IDN, body, 0)
        for m in range(NCHUNK - RN, NCHUNK):
            scatter_wait(m, m % RN, m % IDN)
        plsc.subcore_barrier()
        pltpu.sync_copy(acc.at[pl.ds(row0, RPT)],
                        out_hbm.at[c].at[pl.ds(row0, RPT)])

        @pl.when(s == NS - 1)
        def _():
            pltpu.sync_copy(acc.at[pl.ds(TAIL0, TAIL)],
                            out_hbm.at[c].at[pl.ds(TAIL0, TAIL)])

    return agg


def _dense1(p, W1a, b1a, W1b, b1b, g1, be1):
    def body(p_ref, wa, ba, wb, bb, gg, bb2, out_ref):
        z = p_ref[0] + p_ref[1]
        z = jnp.dot(z, wa[...], preferred_element_type=jnp.float32) + ba[...]
        z = jnp.maximum(z, 0.0)
        z = jnp.dot(z, wb[...], preferred_element_type=jnp.float32) + bb[...]
        h = _selu(z)
        mean = jnp.mean(h, axis=0, keepdims=True)
        var = jnp.mean((h - mean) ** 2, axis=0, keepdims=True)
        out_ref[...] = gg[...] * (h - mean) * lax.rsqrt(var + BN_EPS) + bb2[...]

    return pl.pallas_call(
        body,
        out_shape=jax.ShapeDtypeStruct((N, HID), jnp.float32),
    )(p, W1a, b1a, W1b, b1b, g1, be1)


def _dense2(p, W2a, b2a, W2b, b2b, g2, be2):
    def body(p_ref, wa, ba, wb, bb, gg, bb2, out_ref):
        z = p_ref[0] + p_ref[1]
        z = jnp.dot(z, wa[...], preferred_element_type=jnp.float32) + ba[...]
        z = jnp.maximum(z, 0.0)
        z = jnp.dot(z, wb[...], preferred_element_type=jnp.float32) + bb[...]
        h2 = _selu(z)
        mean = jnp.mean(h2, axis=0, keepdims=True)
        var = jnp.mean((h2 - mean) ** 2, axis=0, keepdims=True)
        h2 = gg[...] * (h2 - mean) * lax.rsqrt(var + BN_EPS) + bb2[...]
        m = jnp.max(h2, axis=1, keepdims=True)
        e = jnp.exp(h2 - m)
        out_ref[...] = e / jnp.sum(e, axis=1, keepdims=True)

    return pl.pallas_call(
        body,
        out_shape=jax.ShapeDtypeStruct((N, NCLS), jnp.float32),
    )(p, W2a, b2a, W2b, b2b, g2, be2)


def kernel(x, edge_index, W1a, b1a, W1b, b1b, bn1_g, bn1_b,
           W2a, b2a, W2b, b2b, bn2_g, bn2_b,
           g, A_k, D, Kindices, de, M, I):
    ei = edge_index.astype(jnp.int32)
    srcf = ei[0]
    dstf = ei[1]
    zeros = jnp.zeros((N, HID), dtype=jnp.float32)

    agg = _make_agg(HID)

    b1a_ = b1a.reshape(1, HID)
    b1b_ = b1b.reshape(1, HID)
    g1_ = bn1_g.reshape(1, HID)
    be1_ = bn1_b.reshape(1, HID)
    b2a_ = b2a.reshape(1, HID)
    b2b_ = b2b.reshape(1, NCLS)
    g2_ = bn2_g.reshape(1, NCLS)
    be2_ = bn2_b.reshape(1, NCLS)

    p = agg(x, srcf, dstf, zeros)
    h = _dense1(p, W1a, b1a_, W1b, b1b_, g1_, be1_)
    p2 = agg(h, srcf, dstf, zeros)
    out = _dense2(p2, W2a, b2a_, W2b, b2b_, g2_, be2_)
    return out
